# baseline (speedup 1.0000x reference)
import functools
import jax
import jax.numpy as jnp
from jax import lax
from jax.experimental import pallas as pl
from jax.experimental.pallas import tpu as pltpu
from jax.experimental.pallas import tpu_sc as plsc

N_FEAT = 1000000
N_DIM = 32
BATCH = 16384

_info = plsc.get_sparse_core_info()
_NC = _info.num_cores
_NS = _info.num_subcores
_NW = _NC * _NS
_BPW = BATCH // _NW

_mesh = plsc.VectorSubcoreMesh(core_axis_name="c", subcore_axis_name="s")


@functools.partial(
    pl.kernel,
    mesh=_mesh,
    out_type=(
        jax.ShapeDtypeStruct((BATCH,), jnp.float32),
        jax.ShapeDtypeStruct((4, BATCH // 128, 8, 128), jnp.float32),
    ),
    scratch_types=[
        pltpu.VMEM((_BPW,), jnp.int32),
        pltpu.VMEM((_BPW,), jnp.float32),
        pltpu.VMEM((N_DIM, 4, 128), jnp.float32),
        pltpu.SemaphoreType.DMA,
        pltpu.SemaphoreType.DMA,
        pltpu.SemaphoreType.DMA,
    ],
    compiler_params=pltpu.CompilerParams(use_tc_tiling_on_sc=False),
)
def _lookup(idx_hbm, vt_hbm, biasf_hbm, bias_out, out4,
            idx_v, bias_v, cols3_v, sem_v, sem_b, sem_o):
    wid = lax.axis_index("s") * _NC + lax.axis_index("c")
    base = wid * _BPW
    pltpu.sync_copy(idx_hbm.at[pl.ds(base, _BPW)], idx_v)
    cb = pltpu.async_copy(biasf_hbm.at[idx_v], bias_v, sem_b)
    gathers = []
    for d in range(N_DIM):
        for jj in range(4):
            gathers.append(pltpu.async_copy(
                vt_hbm.at[d].at[idx_v.at[pl.ds(jj * 128, 128)]],
                cols3_v.at[d, jj], sem_v))
    for gth in gathers:
        gth.wait()
    copies = []
    for d in range(N_DIM):
        g, r = d // 8, d % 8
        copies.append(pltpu.async_copy(
            cols3_v.at[d], out4.at[g, pl.ds(4 * wid, 4), r], sem_o))
    for c in copies:
        c.wait()
    cb.wait()
    pltpu.sync_copy(bias_v, bias_out.at[pl.ds(base, _BPW)])


def kernel(index, vect, bias):
    idx = index.astype(jnp.int32)
    bias_out, out4 = _lookup(idx, vect.T, bias[:, 0])
    return (bias_out.reshape(BATCH, 1),
            out4.transpose(1, 3, 0, 2).reshape(BATCH, N_DIM))


# speedup vs baseline: 6.6042x; 6.6042x over previous
import functools
import jax
import jax.numpy as jnp
from jax import lax
from jax.experimental import pallas as pl
from jax.experimental.pallas import tpu as pltpu
from jax.experimental.pallas import tpu_sc as plsc

N_FEAT = 1000000
N_DIM = 32
BATCH = 16384

_info = plsc.get_sparse_core_info()
_NC = _info.num_cores
_NS = _info.num_subcores
_NW = _NC * _NS
_BPW = BATCH // _NW
_CH = 64
_NCH = _BPW // _CH
_NB = (N_FEAT + 127) // 128

_mesh = plsc.VectorSubcoreMesh(core_axis_name="c", subcore_axis_name="s")


@functools.partial(
    pl.kernel,
    mesh=_mesh,
    out_type=(
        jax.ShapeDtypeStruct((BATCH,), jnp.float32),
        jax.ShapeDtypeStruct((N_DIM, BATCH), jnp.float32),
    ),
    scratch_types=[
        pltpu.VMEM((_BPW,), jnp.int32),
        pltpu.VMEM((_BPW,), jnp.int32),
        pltpu.VMEM((_BPW,), jnp.float32),
        pltpu.VMEM((_BPW // 2, 128), jnp.float32),
        pltpu.VMEM((_CH * 8, N_DIM), jnp.float32),
        pltpu.VMEM((N_DIM, _BPW), jnp.float32),
        pltpu.SemaphoreType.DMA,
        pltpu.SemaphoreType.DMA,
    ],
    compiler_params=pltpu.CompilerParams(
        use_tc_tiling_on_sc=True, needs_layout_passes=False),
)
def _lookup(idx_hbm, vc_hbm, biasp_hbm, bias_out, outT,
            idx_v, blk_v, bias_v, bfetch_v, vfetch_v, cols_v, sem_g, sem_t):
    wid = lax.axis_index("s") * _NC + lax.axis_index("c")
    base = wid * _BPW
    pltpu.sync_copy(idx_hbm.at[pl.ds(base, _BPW)], idx_v)
    lanes = lax.iota(jnp.int32, 16)
    nvec = _BPW // 16

    def bias_blk(jb):
        pos = jb * 16 + lanes
        i16 = plsc.load_gather(idx_v, [pos])
        plsc.store_scatter(blk_v, [pos], lax.shift_right_logical(i16, 7))

    pl.loop(0, nvec)(bias_blk)
    for h in range(2):
        pltpu.async_copy(
            biasp_hbm.at[blk_v.at[pl.ds(h * 256, 256)]], bfetch_v,
            sem_g).wait()

        def bias_ext(jb, _h=h):
            pos = jb * 16 + lanes
            gpos = _h * 256 + pos
            i16 = plsc.load_gather(idx_v, [gpos])
            col = lax.bitwise_and(i16, 127)
            vals = plsc.load_gather(bfetch_v, [pos, col])
            plsc.store_scatter(bias_v, [gpos], vals)

        pl.loop(0, 256 // 16)(bias_ext)

    for ch in range(_NCH):

        def fetch(jb, _ch=ch):
            v16 = idx_v[pl.ds(_ch * _CH + jb * 16, 16)]
            for l in range(16):
                i = v16[l]
                t8 = pl.multiple_of((i // 8) * 8, 8)
                pltpu.async_copy(
                    vc_hbm.at[pl.ds(t8, 8)],
                    vfetch_v.at[pl.ds((jb * 16 + l) * 8, 8)], sem_t)

        pl.loop(0, _CH // 16)(fetch)
        pltpu.make_async_copy(
            vc_hbm.at[pl.ds(0, _CH * 8)], vfetch_v, sem_t).wait()

        for jb in range(_CH // 16):
            pos = jb * 16 + lanes
            gpos = ch * _CH + pos
            i16 = plsc.load_gather(idx_v, [gpos])
            rowid = pos * 8 + lax.bitwise_and(i16, 7)

            def dbody(d, _rowid=rowid, _gpos=gpos):
                vals = plsc.load_gather(
                    vfetch_v, [_rowid, lax.broadcast(d, (16,))])
                plsc.store_scatter(
                    cols_v, [lax.broadcast(d, (16,)), _gpos], vals)

            pl.loop(0, N_DIM)(dbody)

    pltpu.sync_copy(cols_v, outT.at[:, pl.ds(base, _BPW)])
    pltpu.sync_copy(bias_v, bias_out.at[pl.ds(base, _BPW)])


def kernel(index, vect, bias):
    idx = index.astype(jnp.int32)
    biasp = jnp.pad(bias[:, 0], (0, _NB * 128 - N_FEAT)).reshape(_NB, 128)
    bias_out, outT = _lookup(idx, vect, biasp)
    return bias_out.reshape(BATCH, 1), outT.T


# speedup vs baseline: 11.4284x; 1.7305x over previous
import functools
import jax
import jax.numpy as jnp
from jax import lax
from jax.experimental import pallas as pl
from jax.experimental.pallas import tpu as pltpu
from jax.experimental.pallas import tpu_sc as plsc

N_FEAT = 1000000
N_DIM = 32
BATCH = 16384

_info = plsc.get_sparse_core_info()
_NC = _info.num_cores
_NS = _info.num_subcores
_NW = _NC * _NS
_BPW = BATCH // _NW
_CH = 64
_NCH = _BPW // _CH
_NB = (N_FEAT + 127) // 128

_mesh = plsc.VectorSubcoreMesh(core_axis_name="c", subcore_axis_name="s")


@functools.partial(
    pl.kernel,
    mesh=_mesh,
    out_type=(
        jax.ShapeDtypeStruct((BATCH,), jnp.float32),
        jax.ShapeDtypeStruct((N_DIM, BATCH), jnp.float32),
    ),
    scratch_types=[
        pltpu.VMEM((_BPW,), jnp.int32),
        pltpu.VMEM((_BPW,), jnp.int32),
        pltpu.VMEM((_BPW,), jnp.float32),
        pltpu.VMEM((_BPW // 2, 128), jnp.float32),
        pltpu.VMEM((_CH * 8, N_DIM), jnp.float32),
        pltpu.VMEM((N_DIM, _BPW), jnp.float32),
        pltpu.SemaphoreType.DMA,
        pltpu.SemaphoreType.DMA,
    ],
    compiler_params=pltpu.CompilerParams(
        use_tc_tiling_on_sc=True, needs_layout_passes=False),
)
def _lookup(idx_hbm, vc_hbm, biasp_hbm, bias_out, outT,
            idx_v, blk_v, bias_v, bfetch_v, vfetch_v, cols_v, sem_g, sem_t):
    wid = lax.axis_index("s") * _NC + lax.axis_index("c")
    base = wid * _BPW
    pltpu.sync_copy(idx_hbm.at[pl.ds(base, _BPW)], idx_v)
    lanes = lax.iota(jnp.int32, 16)
    nvec = _BPW // 16

    def bias_blk(jb):
        pos = jb * 16 + lanes
        i16 = plsc.load_gather(idx_v, [pos])
        plsc.store_scatter(blk_v, [pos], lax.shift_right_logical(i16, 7))

    pl.loop(0, nvec)(bias_blk)
    for h in range(2):
        pltpu.async_copy(
            biasp_hbm.at[blk_v.at[pl.ds(h * 256, 256)]], bfetch_v,
            sem_g).wait()

        def bias_ext(jb, _h=h):
            pos = jb * 16 + lanes
            gpos = _h * 256 + pos
            i16 = plsc.load_gather(idx_v, [gpos])
            col = lax.bitwise_and(i16, 127)
            vals = plsc.load_gather(bfetch_v, [pos, col])
            plsc.store_scatter(bias_v, [gpos], vals)

        pl.loop(0, 256 // 16)(bias_ext)

    for ch in range(_NCH):

        def fetch(jb, _ch=ch):
            v16 = idx_v[pl.ds(_ch * _CH + jb * 16, 16)]
            for l in range(16):
                i = v16[l]
                pltpu.async_copy(
                    vc_hbm.at[i // 8],
                    vfetch_v.at[pl.ds((jb * 16 + l) * 8, 8)], sem_t)

        pl.loop(0, _CH // 16)(fetch)
        pltpu.make_async_copy(
            vc_hbm.at[pl.ds(0, _CH)], vfetch_v.reshape(_CH, 8, N_DIM),
            sem_t).wait()

        for jb in range(_CH // 16):
            pos = jb * 16 + lanes
            gpos = ch * _CH + pos
            i16 = plsc.load_gather(idx_v, [gpos])
            rowid = pos * 8 + lax.bitwise_and(i16, 7)

            def dbody(d, _rowid=rowid, _gpos=gpos):
                vals = plsc.load_gather(
                    vfetch_v, [_rowid, lax.broadcast(d, (16,))])
                plsc.store_scatter(
                    cols_v, [lax.broadcast(d, (16,)), _gpos], vals)

            pl.loop(0, N_DIM)(dbody)

    pltpu.sync_copy(cols_v, outT.at[:, pl.ds(base, _BPW)])
    pltpu.sync_copy(bias_v, bias_out.at[pl.ds(base, _BPW)])


def kernel(index, vect, bias):
    idx = index.astype(jnp.int32)
    biasp = jnp.pad(bias[:, 0], (0, _NB * 128 - N_FEAT)).reshape(_NB, 128)
    bias_out, outT = _lookup(idx, vect.reshape(N_FEAT // 8, 8, N_DIM), biasp)
    return bias_out.reshape(BATCH, 1), outT.T


# speedup vs baseline: 11.8852x; 1.0400x over previous
import functools
import jax
import jax.numpy as jnp
from jax import lax
from jax.experimental import pallas as pl
from jax.experimental.pallas import tpu as pltpu
from jax.experimental.pallas import tpu_sc as plsc

N_FEAT = 1000000
N_DIM = 32
BATCH = 16384

_info = plsc.get_sparse_core_info()
_NC = _info.num_cores
_NS = _info.num_subcores
_NW = _NC * _NS
_BPW = BATCH // _NW
_CH = 32
_NCH = _BPW // _CH
_NB = (N_FEAT + 127) // 128

_mesh = plsc.VectorSubcoreMesh(core_axis_name="c", subcore_axis_name="s")


@functools.partial(
    pl.kernel,
    mesh=_mesh,
    out_type=(
        jax.ShapeDtypeStruct((BATCH,), jnp.float32),
        jax.ShapeDtypeStruct((N_DIM, BATCH), jnp.float32),
    ),
    scratch_types=[
        pltpu.VMEM((_BPW,), jnp.int32),
        pltpu.VMEM((_BPW,), jnp.int32),
        pltpu.VMEM((_BPW,), jnp.float32),
        pltpu.VMEM((_BPW // 2, 128), jnp.float32),
        pltpu.VMEM((_CH * 8, N_DIM), jnp.float32),
        pltpu.VMEM((_CH * 8, N_DIM), jnp.float32),
        pltpu.VMEM((N_DIM, _BPW), jnp.float32),
        pltpu.SemaphoreType.DMA,
        pltpu.SemaphoreType.DMA,
        pltpu.SemaphoreType.DMA,
    ],
    compiler_params=pltpu.CompilerParams(
        use_tc_tiling_on_sc=True, needs_layout_passes=False),
)
def _lookup(idx_hbm, vc_hbm, biasp_hbm, bias_out, outT,
            idx_v, blk_v, bias_v, bfetch_v, vfetch_a, vfetch_b, cols_v,
            sem_g, sem_ta, sem_tb):
    wid = lax.axis_index("s") * _NC + lax.axis_index("c")
    base = wid * _BPW
    pltpu.sync_copy(idx_hbm.at[pl.ds(base, _BPW)], idx_v)
    lanes = lax.iota(jnp.int32, 16)
    nvec = _BPW // 16

    def bias_blk(jb):
        pos = jb * 16 + lanes
        i16 = plsc.load_gather(idx_v, [pos])
        plsc.store_scatter(blk_v, [pos], lax.shift_right_logical(i16, 7))

    pl.loop(0, nvec)(bias_blk)
    for h in range(2):
        pltpu.async_copy(
            biasp_hbm.at[blk_v.at[pl.ds(h * 256, 256)]], bfetch_v,
            sem_g).wait()

        def bias_ext(jb, _h=h):
            pos = jb * 16 + lanes
            gpos = _h * 256 + pos
            i16 = plsc.load_gather(idx_v, [gpos])
            col = lax.bitwise_and(i16, 127)
            vals = plsc.load_gather(bfetch_v, [pos, col])
            plsc.store_scatter(bias_v, [gpos], vals)

        pl.loop(0, 256 // 16)(bias_ext)

    bufs = (vfetch_a, vfetch_b)
    sems = (sem_ta, sem_tb)

    def issue(ch):
        buf, sem = bufs[ch % 2], sems[ch % 2]

        def fetch(jb, _ch=ch, _buf=buf, _sem=sem):
            v16 = idx_v[pl.ds(_ch * _CH + jb * 16, 16)]
            for l in range(16):
                i = v16[l]
                pltpu.async_copy(
                    vc_hbm.at[i // 8],
                    _buf.at[pl.ds((jb * 16 + l) * 8, 8)], _sem)

        pl.loop(0, _CH // 16)(fetch)

    issue(0)
    for ch in range(_NCH):
        if ch + 1 < _NCH:
            issue(ch + 1)
        buf, sem = bufs[ch % 2], sems[ch % 2]
        pltpu.make_async_copy(
            vc_hbm.at[pl.ds(0, _CH)], buf.reshape(_CH, 8, N_DIM),
            sem).wait()

        for jb in range(_CH // 16):
            pos = jb * 16 + lanes
            gpos = ch * _CH + pos
            i16 = plsc.load_gather(idx_v, [gpos])
            rowid = pos * 8 + lax.bitwise_and(i16, 7)

            def dbody(d, _rowid=rowid, _gpos=gpos, _buf=buf):
                vals = plsc.load_gather(
                    _buf, [_rowid, lax.broadcast(d, (16,))])
                plsc.store_scatter(
                    cols_v, [lax.broadcast(d, (16,)), _gpos], vals)

            pl.loop(0, N_DIM)(dbody)

    pltpu.sync_copy(cols_v, outT.at[:, pl.ds(base, _BPW)])
    pltpu.sync_copy(bias_v, bias_out.at[pl.ds(base, _BPW)])


def kernel(index, vect, bias):
    idx = index.astype(jnp.int32)
    biasp = jnp.pad(bias[:, 0], (0, _NB * 128 - N_FEAT)).reshape(_NB, 128)
    bias_out, outT = _lookup(idx, vect.reshape(N_FEAT // 8, 8, N_DIM), biasp)
    return bias_out.reshape(BATCH, 1), outT.T
